# Initial kernel scaffold; baseline (speedup 1.0000x reference)
import jax
import jax.numpy as jnp
from jax.experimental import pallas as pl


def kernel(pos, atomic_numbers, mask, emb, t_w, W1_0, b1_0, W2_0, b2_0, Wx_0, W3_0, b3_0, W1_1, b1_1, W2_1, b2_1, Wx_1, W3_1, b3_1, t, noise_raw):
    raise NotImplementedError("write your pallas kernel here")


# speedup vs baseline: 8.1989x; 8.1989x over previous
import functools

import jax
import jax.numpy as jnp
from jax.experimental import pallas as pl

B = 256
NMAX = 48
N = B * NMAX
HID = 64
NTYPES = 100
SIGMA_MIN = 0.01
SIGMA_MAX = 10.0
G = 8


def _silu(v):
    return v * jax.nn.sigmoid(v)


def _vesde_block(pos_ref, an_ref, tn_ref, noise_ref, emb_ref, t_w_ref,
                 *rest):
    w_refs = rest[:-1]
    out_ref = rest[-1]
    pos = pos_ref[...]
    noise_raw = noise_ref[...]
    tn = tn_ref[...]
    an = an_ref[...]

    mean = jnp.mean(noise_raw, axis=1, keepdims=True)
    noise = noise_raw - mean
    std = SIGMA_MIN * jnp.exp(tn * jnp.log(SIGMA_MAX / SIGMA_MIN))
    perturbed = pos + noise * std

    onehot = (jax.lax.broadcasted_iota(jnp.int32, (G, NMAX, NTYPES), 2)
              == an).astype(jnp.float32)
    h_emb = jnp.dot(onehot.reshape(G * NMAX, NTYPES), emb_ref[...],
                    preferred_element_type=jnp.float32, precision=jax.lax.Precision.HIGHEST).reshape(G, NMAX, HID)
    h = h_emb + tn * t_w_ref[...].reshape(1, 1, HID)

    x = perturbed
    diag = (jax.lax.broadcasted_iota(jnp.int32, (NMAX, NMAX), 0)
            == jax.lax.broadcasted_iota(jnp.int32, (NMAX, NMAX), 1))
    notdiag = jnp.where(diag, 0.0, 1.0).reshape(1, NMAX, NMAX, 1)

    for l in range(2):
        (W1a, W1b, w1c, b1, W2, b2, Wx, W3a, W3b, b3) = (
            r[...] for r in w_refs[l * 10:(l + 1) * 10])
        rel4 = x[:, :, None, :] - x[:, None, :, :]
        d2 = jnp.sum(rel4 * rel4, axis=-1)

        h2 = h.reshape(G * NMAX, HID)
        A = jnp.dot(h2, W1a, preferred_element_type=jnp.float32, precision=jax.lax.Precision.HIGHEST)
        Bv = jnp.dot(h2, W1b, preferred_element_type=jnp.float32, precision=jax.lax.Precision.HIGHEST)
        A = A.reshape(G, NMAX, HID)
        Bv = Bv.reshape(G, NMAX, HID)
        pre1 = (A[:, :, None, :] + Bv[:, None, :, :]
                + d2[:, :, :, None] * w1c.reshape(1, 1, 1, HID)
                + b1.reshape(1, 1, 1, HID))
        t1 = _silu(pre1).reshape(G * NMAX * NMAX, HID)
        m = _silu(jnp.dot(t1, W2, preferred_element_type=jnp.float32, precision=jax.lax.Precision.HIGHEST)
                  + b2.reshape(1, HID))
        coef = jnp.dot(m, Wx, preferred_element_type=jnp.float32, precision=jax.lax.Precision.HIGHEST)
        m4 = m.reshape(G, NMAX, NMAX, HID)

        agg = jnp.sum(m4 * notdiag, axis=1)
        agg2 = agg.reshape(G * NMAX, HID)
        pre3 = (jnp.dot(h2, W3a, preferred_element_type=jnp.float32, precision=jax.lax.Precision.HIGHEST)
                + jnp.dot(agg2, W3b, preferred_element_type=jnp.float32, precision=jax.lax.Precision.HIGHEST)
                + b3.reshape(1, HID))
        h = h + _silu(pre3).reshape(G, NMAX, HID)

        c4 = coef.reshape(G, NMAX, NMAX, 1)
        xagg = jnp.sum(c4 * x[:, :, None, :], axis=1)
        csum = jnp.sum(c4, axis=1)
        x = x + (xagg - x * csum) * (1.0 / (NMAX - 1))

    resid = x - perturbed + noise
    block_loss = jnp.sum(resid * resid).reshape(1, 1)

    @pl.when(pl.program_id(0) == 0)
    def _():
        out_ref[...] = jnp.zeros((1, 1), jnp.float32)
    out_ref[...] += block_loss


def kernel(pos, atomic_numbers, mask, emb, t_w, W1_0, b1_0, W2_0, b2_0,
           Wx_0, W3_0, b3_0, W1_1, b1_1, W2_1, b2_1, Wx_1, W3_1, b3_1,
           t, noise_raw):
    Bm, nmax = mask.shape
    pos3 = pos.reshape(Bm, nmax, 3)
    noise3 = noise_raw.reshape(Bm, nmax, 3)
    an3 = atomic_numbers.reshape(Bm, nmax, 1).astype(jnp.int32)
    tn3 = jnp.broadcast_to(t.reshape(Bm, 1, 1), (Bm, nmax, 1))

    def split_w(W1, b1, W2, b2, Wx, W3, b3):
        return (W1[:HID], W1[HID:2 * HID], W1[2 * HID].reshape(1, HID),
                b1.reshape(1, HID), W2, b2.reshape(1, HID), Wx,
                W3[:HID], W3[HID:], b3.reshape(1, HID))

    weights = (split_w(W1_0, b1_0, W2_0, b2_0, Wx_0, W3_0, b3_0)
               + split_w(W1_1, b1_1, W2_1, b2_1, Wx_1, W3_1, b3_1))

    grid = Bm // G
    blk = lambda *shape: pl.BlockSpec(shape, lambda i: (i,) + (0,) * (len(shape) - 1))
    full = lambda a: pl.BlockSpec(a.shape, lambda i: (0,) * a.ndim)

    in_specs = [
        blk(G, nmax, 3),
        blk(G, nmax, 1),
        blk(G, nmax, 1),
        blk(G, nmax, 3),
        full(emb),
        full(t_w.reshape(1, HID)),
    ] + [full(w) for w in weights]

    out = pl.pallas_call(
        _vesde_block,
        grid=(grid,),
        in_specs=in_specs,
        out_specs=pl.BlockSpec((1, 1), lambda i: (0, 0)),
        out_shape=jax.ShapeDtypeStruct((1, 1), jnp.float32),
    )(pos3, an3, tn3, noise3, emb, t_w.reshape(1, HID), *weights)
    return out[0, 0] / N


# speedup vs baseline: 31.5393x; 3.8468x over previous
import functools

import jax
import jax.numpy as jnp
from jax.experimental import pallas as pl

B = 256
NMAX = 48
N = B * NMAX
HID = 64
NTYPES = 100
SIGMA_MIN = 0.01
SIGMA_MAX = 10.0
G = 8


def _silu(v):
    return v * jax.nn.sigmoid(v)


def _b16(v):
    return v.astype(jnp.bfloat16)


def _dot16(a, b):
    return jnp.dot(_b16(a), _b16(b), preferred_element_type=jnp.float32)


def _vesde_block(pos_ref, an_ref, tn_ref, noise_ref, emb_ref, t_w_ref,
                 *rest):
    w_refs = rest[:-1]
    out_ref = rest[-1]
    pos = pos_ref[...]
    noise_raw = noise_ref[...]
    tn = tn_ref[...]
    an = an_ref[...]

    mean = (jnp.sum(noise_raw, axis=1, keepdims=True)
            / jnp.float32(NMAX))
    noise = noise_raw - mean
    std = SIGMA_MIN * jnp.exp(tn * jnp.log(SIGMA_MAX / SIGMA_MIN))
    perturbed = pos + noise * std

    onehot = (jax.lax.broadcasted_iota(jnp.int32, (G, NMAX, NTYPES), 2)
              == an).astype(jnp.float32)
    h_emb = jnp.dot(onehot.reshape(G * NMAX, NTYPES), emb_ref[...],
                    preferred_element_type=jnp.float32, precision=jax.lax.Precision.HIGHEST).reshape(G, NMAX, HID)
    h = h_emb + tn * t_w_ref[...].reshape(1, 1, HID)

    x = perturbed
    diag = (jax.lax.broadcasted_iota(jnp.int32, (NMAX, NMAX), 0)
            == jax.lax.broadcasted_iota(jnp.int32, (NMAX, NMAX), 1))
    notdiag = jnp.where(diag, 0.0, 1.0).reshape(1, NMAX, NMAX, 1)

    for l in range(2):
        (W1a, W1b, w1c, b1, W2, b2, Wx, W3a, W3b, b3) = (
            r[...] for r in w_refs[l * 10:(l + 1) * 10])
        rel4 = x[:, :, None, :] - x[:, None, :, :]
        d24 = jnp.sum(rel4 * rel4, axis=-1, keepdims=True)

        h2 = h.reshape(G * NMAX, HID)
        A = _dot16(h2, W1a).reshape(G, NMAX, HID)
        Bv = _dot16(h2, W1b).reshape(G, NMAX, HID)
        d2term = (_b16(d24).astype(jnp.float32)
                  * _b16(w1c).astype(jnp.float32).reshape(1, 1, 1, HID))
        pre1 = (A[:, :, None, :] + Bv[:, None, :, :]
                + d2term + b1.reshape(1, 1, 1, HID))
        t1 = _silu(pre1).reshape(G * NMAX * NMAX, HID)
        m = _silu(_dot16(t1, W2) + b2.reshape(1, HID))
        coef = _dot16(m, Wx)
        m4 = m.reshape(G, NMAX, NMAX, HID)

        agg = jnp.sum(m4 * notdiag, axis=1)
        agg2 = agg.reshape(G * NMAX, HID)
        pre3 = (_dot16(h2, W3a) + _dot16(agg2, W3b)
                + b3.reshape(1, HID))
        h = h + _silu(pre3).reshape(G, NMAX, HID)

        c4 = coef.reshape(G, NMAX, NMAX, 1)
        xupd = jnp.sum(rel4 * c4, axis=1)
        x = x + xupd / jnp.float32(NMAX - 1)

    resid = x - perturbed + noise
    block_loss = jnp.sum(resid * resid).reshape(1, 1)

    @pl.when(pl.program_id(0) == 0)
    def _():
        out_ref[...] = jnp.zeros((1, 1), jnp.float32)
    out_ref[...] += block_loss


def kernel(pos, atomic_numbers, mask, emb, t_w, W1_0, b1_0, W2_0, b2_0,
           Wx_0, W3_0, b3_0, W1_1, b1_1, W2_1, b2_1, Wx_1, W3_1, b3_1,
           t, noise_raw):
    Bm, nmax = mask.shape
    pos3 = pos.reshape(Bm, nmax, 3)
    noise3 = noise_raw.reshape(Bm, nmax, 3)
    an3 = atomic_numbers.reshape(Bm, nmax, 1).astype(jnp.int32)
    tn3 = jnp.broadcast_to(t.reshape(Bm, 1, 1), (Bm, nmax, 1))

    def split_w(W1, b1, W2, b2, Wx, W3, b3):
        return (W1[:HID], W1[HID:2 * HID], W1[2 * HID].reshape(1, HID),
                b1.reshape(1, HID), W2, b2.reshape(1, HID), Wx,
                W3[:HID], W3[HID:], b3.reshape(1, HID))

    weights = (split_w(W1_0, b1_0, W2_0, b2_0, Wx_0, W3_0, b3_0)
               + split_w(W1_1, b1_1, W2_1, b2_1, Wx_1, W3_1, b3_1))

    grid = Bm // G
    blk = lambda *shape: pl.BlockSpec(shape, lambda i: (i,) + (0,) * (len(shape) - 1))
    full = lambda a: pl.BlockSpec(a.shape, lambda i: (0,) * a.ndim)

    in_specs = [
        blk(G, nmax, 3),
        blk(G, nmax, 1),
        blk(G, nmax, 1),
        blk(G, nmax, 3),
        full(emb),
        full(t_w.reshape(1, HID)),
    ] + [full(w) for w in weights]

    out = pl.pallas_call(
        _vesde_block,
        grid=(grid,),
        in_specs=in_specs,
        out_specs=pl.BlockSpec((1, 1), lambda i: (0, 0)),
        out_shape=jax.ShapeDtypeStruct((1, 1), jnp.float32),
    )(pos3, an3, tn3, noise3, emb, t_w.reshape(1, HID), *weights)
    return out[0, 0] / N
